# baseline (speedup 1.0000x reference)
import jax
import jax.numpy as jnp
from jax import lax
from jax.experimental import pallas as pl
from jax.experimental.pallas import tpu as pltpu
from jax.experimental.pallas import tpu_sc as plsc

N = 10000
E = 320000
D = 128
NP = 10112
DP = D // 2

NC = 2
NS = 16
NW = NC * NS
LANES = 16
CPW = DP // NW
CHUNK = 2560
NCHUNKS = E // CHUNK
VPC = CHUNK // LANES
NGRP = VPC // 16
SPILL_SLOTS = 384
NEGINF2 = -8323200


def _pmax(a, b):
    return plsc.bitcast(
        jnp.maximum(plsc.bitcast(a, jnp.bfloat16),
                    plsc.bitcast(b, jnp.bfloat16)), jnp.int32)


def _lost(rv, hv):
    return _pmax(rv, hv) != rv


def _seg_max_body(hT_hbm, src_hbm, dst_hbm, aggT_hbm,
                  hT_v, acc0, acc1,
                  src_a, dst_a, src_b, dst_b,
                  msk_buf, cnt_buf, sp_src, sp_dst, sp_msk,
                  ws_ref, sem_sa, sem_da, sem_sb, sem_db):
    wid = lax.axis_index("s") * NC + lax.axis_index("c")
    iota = lax.iota(jnp.int32, LANES)
    srcs = (src_a, src_b)
    dsts = (dst_a, dst_b)
    sems = ((sem_sa, sem_da), (sem_sb, sem_db))

    def start(k, p):
        pltpu.async_copy(src_hbm.at[pl.ds(k * CHUNK, CHUNK)], srcs[p],
                         sems[p][0])
        pltpu.async_copy(dst_hbm.at[pl.ds(k * CHUNK, CHUNK)], dsts[p],
                         sems[p][1])

    def wait(p):
        pltpu.make_async_copy(src_hbm.at[pl.ds(0, CHUNK)], srcs[p],
                              sems[p][0]).wait()
        pltpu.make_async_copy(dst_hbm.at[pl.ds(0, CHUNK)], dsts[p],
                              sems[p][1]).wait()

    start(0, 0)

    pltpu.sync_copy(hT_hbm.at[pl.ds(wid * (CPW * NP), CPW * NP)], hT_v)

    ws_ref[0] = jnp.int32(0)

    def init_body(i, _):
        v = jnp.full((LANES,), NEGINF2, jnp.int32)
        acc0[pl.ds(i * LANES, LANES)] = v
        acc1[pl.ds(i * LANES, LANES)] = v
        return 0
    lax.fori_loop(0, CPW * NP // LANES, init_body, 0)

    def process_pair(sv, dv, j0):
        sl0 = pl.ds(j0 * LANES, LANES)
        sl1 = pl.ds(j0 * LANES + LANES, LANES)
        s0, d0 = sv[sl0], dv[sl0]
        s1, d1 = sv[sl1], dv[sl1]
        offs = [jnp.full((LANES,), c * NP, jnp.int32) for c in range(CPW)]
        hv0 = [plsc.load_gather(hT_v, [s0 + offs[c]]) for c in range(CPW)]
        hv1 = [plsc.load_gather(hT_v, [s1 + offs[c]]) for c in range(CPW)]
        av0 = [plsc.load_gather(acc0, [d0 + offs[c]]) for c in range(CPW)]
        av1 = [plsc.load_gather(acc1, [d1 + offs[c]]) for c in range(CPW)]
        for c in range(CPW):
            plsc.store_scatter(acc0, [d0 + offs[c]], _pmax(av0[c], hv0[c]))
            plsc.store_scatter(acc1, [d1 + offs[c]], _pmax(av1[c], hv1[c]))
        rv0 = [plsc.load_gather(acc0, [d0 + offs[c]]) for c in range(CPW)]
        rv1 = [plsc.load_gather(acc1, [d1 + offs[c]]) for c in range(CPW)]
        lose0 = _lost(rv0[0], hv0[0]) | _lost(rv0[1], hv0[1])
        lose1 = _lost(rv1[0], hv1[0]) | _lost(rv1[1], hv1[1])
        msk_buf[sl0] = lose0.astype(jnp.int32)
        msk_buf[sl1] = lose1.astype(jnp.int32)
        cnt_buf[sl0] = plsc.all_reduce_population_count(lose0)
        cnt_buf[sl1] = plsc.all_reduce_population_count(lose1)

    def process_masked(s, d, m):
        hvs, rvs = [], []
        for c in range(CPW):
            off = jnp.full((LANES,), c * NP, jnp.int32)
            hv = plsc.load_gather(hT_v, [s + off])
            av = plsc.load_gather(acc0, [d + off])
            plsc.store_scatter(acc0, [d + off], _pmax(av, hv), mask=m)
            hvs.append(hv)
            rvs.append(plsc.load_gather(acc0, [d + off]))
        lose = m & (_lost(rvs[0], hvs[0]) | _lost(rvs[1], hvs[1]))
        nl = plsc.all_reduce_population_count(lose)[0]

        @pl.when(nl > 0)
        def _():
            ws = ws_ref[0]
            base = pl.ds(ws * LANES, LANES)
            sp_src[base] = s
            sp_dst[base] = d
            sp_msk[base] = lose.astype(jnp.int32)
            ws_ref[0] = ws + 1

    def process_chunk(p):
        sv, dv = srcs[p], dsts[p]

        def group_body(g, _):
            for q in range(4):
                process_pair(sv, dv, g * 8 + 2 * q)
            return 0
        lax.fori_loop(0, VPC // 8, group_body, 0)

        def scan_body(g, _):
            cv = plsc.load_gather(cnt_buf, [iota * LANES + g * (16 * LANES)])
            nz = plsc.all_reduce_population_count(cv > 0)[0]

            @pl.when(nz > 0)
            def _():
                for b in range(16):
                    @pl.when(cv[b] > 0)
                    def _(b=b):
                        j = g * 16 + b
                        sl = pl.ds(j * LANES, LANES)
                        process_masked(sv[sl], dv[sl], msk_buf[sl] != 0)
            return 0
        lax.fori_loop(0, NGRP, scan_body, 0)

    def outer_body(i, _):
        k0 = i * 2
        start(k0 + 1, 1)
        wait(0)
        process_chunk(0)

        @pl.when(k0 + 2 < NCHUNKS)
        def _():
            start(k0 + 2, 0)
        wait(1)
        process_chunk(1)
        return 0
    lax.fori_loop(0, NCHUNKS // 2, outer_body, 0)

    if NCHUNKS % 2:
        wait(0)
        process_chunk(0)

    def drain_body(ns):
        ws_ref[0] = jnp.int32(0)

        def vreg_body(j, _):
            base = pl.ds(j * LANES, LANES)
            process_masked(sp_src[base], sp_dst[base], sp_msk[base] != 0)
            return 0
        lax.fori_loop(0, ns, vreg_body, 0)
        return ws_ref[0]

    lax.while_loop(lambda c: c > 0, drain_body, ws_ref[0])

    def merge_body(i, _):
        sl = pl.ds(i * LANES, LANES)
        acc0[sl] = _pmax(acc0[sl], acc1[sl])
        return 0
    lax.fori_loop(0, CPW * NP // LANES, merge_body, 0)

    pltpu.sync_copy(acc0, aggT_hbm.at[pl.ds(wid * (CPW * NP), CPW * NP)])


@jax.jit
def _seg_max(hTp, src, dst):
    mesh = plsc.VectorSubcoreMesh(core_axis_name="c", subcore_axis_name="s")
    return pl.kernel(
        _seg_max_body,
        out_type=jax.ShapeDtypeStruct((DP * NP,), jnp.int32),
        mesh=mesh,
        compiler_params=pltpu.CompilerParams(needs_layout_passes=False),
        scratch_types=[
            pltpu.VMEM((CPW * NP,), jnp.int32),
            pltpu.VMEM((CPW * NP,), jnp.int32),
            pltpu.VMEM((CPW * NP,), jnp.int32),
            pltpu.VMEM((CHUNK,), jnp.int32),
            pltpu.VMEM((CHUNK,), jnp.int32),
            pltpu.VMEM((CHUNK,), jnp.int32),
            pltpu.VMEM((CHUNK,), jnp.int32),
            pltpu.VMEM((VPC * LANES,), jnp.int32),
            pltpu.VMEM((VPC * LANES,), jnp.int32),
            pltpu.VMEM((SPILL_SLOTS * LANES,), jnp.int32),
            pltpu.VMEM((SPILL_SLOTS * LANES,), jnp.int32),
            pltpu.VMEM((SPILL_SLOTS * LANES,), jnp.int32),
            pltpu.SMEM((1,), jnp.int32),
            pltpu.SemaphoreType.DMA,
            pltpu.SemaphoreType.DMA,
            pltpu.SemaphoreType.DMA,
            pltpu.SemaphoreType.DMA,
        ],
    )(hTp, src, dst)


def _pack_body(h_ref, id_ref, out_ref):
    hT = lax.dot_general(id_ref[...], h_ref[...], (((1,), (1,)), ((), ())),
                         preferred_element_type=jnp.float32)
    u = lax.bitcast_convert_type(hT.astype(jnp.bfloat16), jnp.uint16)
    lo = u[:DP].astype(jnp.uint32)
    hi = u[DP:].astype(jnp.uint32)
    out_ref[...] = lax.bitcast_convert_type(lo | (hi << 16), jnp.int32)


def _pack_h(h):
    h_pad = jnp.pad(h, ((0, NP - N), (0, 0)))
    ident = jnp.eye(D, dtype=jnp.float32)
    packed = pl.pallas_call(
        _pack_body,
        out_shape=jax.ShapeDtypeStruct((DP, NP), jnp.int32),
    )(h_pad, ident)
    return packed.reshape(-1)


def _unpack_agg(aggP):
    u = lax.bitcast_convert_type(aggP.reshape(DP, NP)[:, :N], jnp.uint32)
    lo = lax.bitcast_convert_type((u & 0xFFFF).astype(jnp.uint16),
                                  jnp.bfloat16)
    hi = lax.bitcast_convert_type((u >> 16).astype(jnp.uint16),
                                  jnp.bfloat16)
    agg = jnp.concatenate([lo, hi], axis=0)
    return agg.T.astype(jnp.float32)


def _mlp_body(h_ref, aggp_ref, w1_ref, w2_ref, b2_ref, id_ref, out_ref):
    ap = aggp_ref[:, :N]
    lo = lax.bitcast_convert_type(ap << 16, jnp.float32)
    hi = lax.bitcast_convert_type(ap & jnp.int32(-65536), jnp.float32)
    aggT = jnp.concatenate([lo, hi], axis=0)
    aggT = jnp.where(aggT < -1e38, 0.0, aggT)
    agg = lax.dot_general(aggT, id_ref[...], (((0,), (0,)), ((), ())),
                          preferred_element_type=jnp.float32)
    pre = h_ref[...] + agg
    hid = lax.dot_general(pre, w1_ref[...], (((1,), (1,)), ((), ())),
                          preferred_element_type=jnp.float32)
    hid = jnp.maximum(hid, 0.0)
    out = lax.dot_general(hid, w2_ref[...], (((1,), (1,)), ((), ())),
                          preferred_element_type=jnp.float32)
    out_ref[...] = out + b2_ref[...]


def _mlp(h, aggP2d, W1, W2, b2, ident):
    return pl.pallas_call(
        _mlp_body,
        out_shape=jax.ShapeDtypeStruct((N, D), jnp.float32),
    )(h, aggP2d, W1, W2, b2, ident)


def kernel(h, edge_index, W1, W2, b2):
    ei = edge_index.astype(jnp.int32)
    src, dst = ei[0], ei[1]
    aggP = _seg_max(_pack_h(h), src, dst).reshape(DP, NP)
    ident = jnp.eye(D, dtype=jnp.float32)
    return _mlp(h, aggP, W1, W2, b2.reshape(1, D), ident)


# speedup vs baseline: 1.0081x; 1.0081x over previous
import jax
import jax.numpy as jnp
from jax import lax
from jax.experimental import pallas as pl
from jax.experimental.pallas import tpu as pltpu
from jax.experimental.pallas import tpu_sc as plsc

N = 10000
E = 320000
D = 128
NP = 10112
DP = D // 2

NC = 2
NS = 16
NW = NC * NS
LANES = 16
CPW = DP // NW
CHUNK = 2560
NCHUNKS = E // CHUNK
VPC = CHUNK // LANES
NGRP = VPC // 16
SPILL_SLOTS = 384
NEGINF2 = -8323200


def _pmax(a, b):
    return plsc.bitcast(
        jnp.maximum(plsc.bitcast(a, jnp.bfloat16),
                    plsc.bitcast(b, jnp.bfloat16)), jnp.int32)


def _lost(rv, hv):
    return _pmax(rv, hv) != rv


def _seg_max_body(hT_hbm, src_hbm, dst_hbm, aggT_hbm,
                  hT_v, acc0, acc1,
                  src_a, dst_a, src_b, dst_b,
                  msk_buf, cnt_buf, sp_src, sp_dst, sp_msk,
                  ws_ref, sem_sa, sem_da, sem_sb, sem_db):
    wid = lax.axis_index("s") * NC + lax.axis_index("c")
    iota = lax.iota(jnp.int32, LANES)
    srcs = (src_a, src_b)
    dsts = (dst_a, dst_b)
    sems = ((sem_sa, sem_da), (sem_sb, sem_db))

    def start(k, p):
        pltpu.async_copy(src_hbm.at[pl.ds(k * CHUNK, CHUNK)], srcs[p],
                         sems[p][0])
        pltpu.async_copy(dst_hbm.at[pl.ds(k * CHUNK, CHUNK)], dsts[p],
                         sems[p][1])

    def wait(p):
        pltpu.make_async_copy(src_hbm.at[pl.ds(0, CHUNK)], srcs[p],
                              sems[p][0]).wait()
        pltpu.make_async_copy(dst_hbm.at[pl.ds(0, CHUNK)], dsts[p],
                              sems[p][1]).wait()

    start(0, 0)

    pltpu.sync_copy(hT_hbm.at[pl.ds(wid * (CPW * NP), CPW * NP)], hT_v)

    ws_ref[0] = jnp.int32(0)

    def init_body(i, _):
        v = jnp.full((LANES,), NEGINF2, jnp.int32)
        acc0[pl.ds(i * LANES, LANES)] = v
        acc1[pl.ds(i * LANES, LANES)] = v
        return 0
    lax.fori_loop(0, CPW * NP // LANES, init_body, 0)

    def process_pair(sv, dv, j0):
        sl0 = pl.ds(j0 * LANES, LANES)
        sl1 = pl.ds(j0 * LANES + LANES, LANES)
        s0, d0 = sv[sl0], dv[sl0]
        s1, d1 = sv[sl1], dv[sl1]
        offs = [jnp.full((LANES,), c * NP, jnp.int32) for c in range(CPW)]
        hv0 = [plsc.load_gather(hT_v, [s0 + offs[c]]) for c in range(CPW)]
        hv1 = [plsc.load_gather(hT_v, [s1 + offs[c]]) for c in range(CPW)]
        av0 = [plsc.load_gather(acc0, [d0 + offs[c]]) for c in range(CPW)]
        av1 = [plsc.load_gather(acc1, [d1 + offs[c]]) for c in range(CPW)]
        for c in range(CPW):
            plsc.store_scatter(acc0, [d0 + offs[c]], _pmax(av0[c], hv0[c]))
            plsc.store_scatter(acc1, [d1 + offs[c]], _pmax(av1[c], hv1[c]))
        rv0 = [plsc.load_gather(acc0, [d0 + offs[c]]) for c in range(CPW)]
        rv1 = [plsc.load_gather(acc1, [d1 + offs[c]]) for c in range(CPW)]
        lose0 = _lost(rv0[0], hv0[0]) | _lost(rv0[1], hv0[1])
        lose1 = _lost(rv1[0], hv1[0]) | _lost(rv1[1], hv1[1])
        msk_buf[sl0] = lose0.astype(jnp.int32)
        msk_buf[sl1] = lose1.astype(jnp.int32)
        cnt_buf[sl0] = plsc.all_reduce_population_count(lose0)
        cnt_buf[sl1] = plsc.all_reduce_population_count(lose1)

    def process_masked(s, d, m):
        hvs, rvs = [], []
        for c in range(CPW):
            off = jnp.full((LANES,), c * NP, jnp.int32)
            hv = plsc.load_gather(hT_v, [s + off])
            av = plsc.load_gather(acc0, [d + off])
            plsc.store_scatter(acc0, [d + off], _pmax(av, hv), mask=m)
            hvs.append(hv)
            rvs.append(plsc.load_gather(acc0, [d + off]))
        lose = m & (_lost(rvs[0], hvs[0]) | _lost(rvs[1], hvs[1]))
        nl = plsc.all_reduce_population_count(lose)[0]

        @pl.when(nl > 0)
        def _():
            ws = ws_ref[0]
            base = pl.ds(ws * LANES, LANES)
            sp_src[base] = s
            sp_dst[base] = d
            sp_msk[base] = lose.astype(jnp.int32)
            ws_ref[0] = ws + 1

    def process_chunk(p):
        sv, dv = srcs[p], dsts[p]

        def group_body(g, _):
            for q in range(4):
                process_pair(sv, dv, g * 8 + 2 * q)
            return 0
        lax.fori_loop(0, VPC // 8, group_body, 0)

        def scan_body(g, _):
            cv = plsc.load_gather(cnt_buf, [iota * LANES + g * (16 * LANES)])
            nz = plsc.all_reduce_population_count(cv > 0)[0]

            @pl.when(nz > 0)
            def _():
                for b in range(16):
                    @pl.when(cv[b] > 0)
                    def _(b=b):
                        j = g * 16 + b
                        sl = pl.ds(j * LANES, LANES)
                        process_masked(sv[sl], dv[sl], msk_buf[sl] != 0)
            return 0
        lax.fori_loop(0, NGRP, scan_body, 0)

    def outer_body(i, _):
        k0 = i * 2
        start(k0 + 1, 1)
        wait(0)
        process_chunk(0)

        @pl.when(k0 + 2 < NCHUNKS)
        def _():
            start(k0 + 2, 0)
        wait(1)
        process_chunk(1)
        return 0
    lax.fori_loop(0, NCHUNKS // 2, outer_body, 0)

    if NCHUNKS % 2:
        wait(0)
        process_chunk(0)

    def drain_body(ns):
        ws_ref[0] = jnp.int32(0)

        def vreg_body(j, _):
            base = pl.ds(j * LANES, LANES)
            process_masked(sp_src[base], sp_dst[base], sp_msk[base] != 0)
            return 0
        lax.fori_loop(0, ns, vreg_body, 0)
        return ws_ref[0]

    lax.while_loop(lambda c: c > 0, drain_body, ws_ref[0])

    def merge_body(i, _):
        sl = pl.ds(i * LANES, LANES)
        acc0[sl] = _pmax(acc0[sl], acc1[sl])
        return 0
    lax.fori_loop(0, CPW * NP // LANES, merge_body, 0)

    pltpu.sync_copy(acc0, aggT_hbm.at[pl.ds(wid * (CPW * NP), CPW * NP)])


@jax.jit
def _seg_max(hTp, src, dst):
    mesh = plsc.VectorSubcoreMesh(core_axis_name="c", subcore_axis_name="s")
    return pl.kernel(
        _seg_max_body,
        out_type=jax.ShapeDtypeStruct((DP * NP,), jnp.int32),
        mesh=mesh,
        compiler_params=pltpu.CompilerParams(needs_layout_passes=False),
        scratch_types=[
            pltpu.VMEM((CPW * NP,), jnp.int32),
            pltpu.VMEM((CPW * NP,), jnp.int32),
            pltpu.VMEM((CPW * NP,), jnp.int32),
            pltpu.VMEM((CHUNK,), jnp.int32),
            pltpu.VMEM((CHUNK,), jnp.int32),
            pltpu.VMEM((CHUNK,), jnp.int32),
            pltpu.VMEM((CHUNK,), jnp.int32),
            pltpu.VMEM((VPC * LANES,), jnp.int32),
            pltpu.VMEM((VPC * LANES,), jnp.int32),
            pltpu.VMEM((SPILL_SLOTS * LANES,), jnp.int32),
            pltpu.VMEM((SPILL_SLOTS * LANES,), jnp.int32),
            pltpu.VMEM((SPILL_SLOTS * LANES,), jnp.int32),
            pltpu.SMEM((1,), jnp.int32),
            pltpu.SemaphoreType.DMA,
            pltpu.SemaphoreType.DMA,
            pltpu.SemaphoreType.DMA,
            pltpu.SemaphoreType.DMA,
        ],
    )(hTp, src, dst)


def _pack_h(h):
    u = lax.bitcast_convert_type(h.astype(jnp.bfloat16), jnp.uint16)
    lo = u[:, :DP].astype(jnp.uint32)
    hi = u[:, DP:].astype(jnp.uint32)
    packed = (lo | (hi << 16)).T
    packed = jnp.pad(packed, ((0, 0), (0, NP - N)))
    return lax.bitcast_convert_type(packed, jnp.int32).reshape(-1)


def _unpack_agg(aggP):
    u = lax.bitcast_convert_type(aggP.reshape(DP, NP)[:, :N], jnp.uint32)
    lo = lax.bitcast_convert_type((u & 0xFFFF).astype(jnp.uint16),
                                  jnp.bfloat16)
    hi = lax.bitcast_convert_type((u >> 16).astype(jnp.uint16),
                                  jnp.bfloat16)
    agg = jnp.concatenate([lo, hi], axis=0)
    return agg.T.astype(jnp.float32)


def _mlp_body(h_ref, aggp_ref, w1_ref, w2_ref, b2_ref, id_ref, out_ref):
    ap = aggp_ref[:, :N]
    lo = lax.bitcast_convert_type(ap << 16, jnp.float32)
    hi = lax.bitcast_convert_type(ap & jnp.int32(-65536), jnp.float32)
    aggT = jnp.concatenate([lo, hi], axis=0)
    aggT = jnp.where(aggT < -1e38, 0.0, aggT)
    agg = lax.dot_general(aggT, id_ref[...], (((0,), (0,)), ((), ())),
                          preferred_element_type=jnp.float32)
    pre = h_ref[...] + agg
    hid = lax.dot_general(pre, w1_ref[...], (((1,), (1,)), ((), ())),
                          preferred_element_type=jnp.float32)
    hid = jnp.maximum(hid, 0.0)
    out = lax.dot_general(hid, w2_ref[...], (((1,), (1,)), ((), ())),
                          preferred_element_type=jnp.float32)
    out_ref[...] = out + b2_ref[...]


def _mlp(h, aggP2d, W1, W2, b2, ident):
    return pl.pallas_call(
        _mlp_body,
        out_shape=jax.ShapeDtypeStruct((N, D), jnp.float32),
    )(h, aggP2d, W1, W2, b2, ident)


def kernel(h, edge_index, W1, W2, b2):
    ei = edge_index.astype(jnp.int32)
    src, dst = ei[0], ei[1]
    aggP = _seg_max(_pack_h(h), src, dst).reshape(DP, NP)
    ident = jnp.eye(D, dtype=jnp.float32)
    return _mlp(h, aggP, W1, W2, b2.reshape(1, D), ident)
